# baseline (speedup 1.0000x reference)
import functools

import jax
import jax.numpy as jnp
from jax import lax
from jax.experimental import pallas as pl
from jax.experimental.pallas import tpu as pltpu
from jax.experimental.pallas import tpu_sc as plsc

N = 10000
E = 160000
IN_FEATS = 128
H1 = 128
H2 = 64

NC = 2
NS = 16
NW = NC * NS
LANES = 16

N_PAD = 10240
R = 512
NBUF = 2
K1, CPT1 = 64, 80
K2, CPT2 = 128, 40
EPT = K1 * CPT1
E_PAD = EPT * NW
assert K2 * CPT2 == EPT

F1E = 144
F2E = 80
ZROWS = 16


def _elu(x):
    return jnp.where(x > 0, x, jnp.exp(jnp.minimum(x, 0.0)) - 1.0)


def _tc_pre_body(f_ref, w_ref, alr_ref, hx_ref, elr_ref, err_ref, mm_ref):
    i = pl.program_id(0)
    fext = hx_ref.shape[1]
    ones_col = fext - LANES
    h = lax.dot_general(f_ref[...], w_ref[...], (((1,), (1,)), ((), ())),
                        preferred_element_type=jnp.float32)
    lane = lax.broadcasted_iota(jnp.int32, h.shape, 1)
    h = h + jnp.where(lane == ones_col, 1.0, 0.0)
    hx_ref[...] = h
    eler = lax.dot_general(h, alr_ref[...], (((1,), (0,)), ((), ())),
                           preferred_element_type=jnp.float32)
    elr_ref[...] = jnp.broadcast_to(eler[:, 0:1], (h.shape[0], LANES))
    err_ref[...] = jnp.broadcast_to(eler[:, 1:2], (h.shape[0], LANES))
    mblk = jnp.max(eler, axis=0, keepdims=True)

    @pl.when(i == 0)
    def _():
        mm_ref[...] = mblk

    @pl.when(i > 0)
    def _():
        mm_ref[...] = jnp.maximum(mm_ref[...], mblk)


def _tc_pre(feat_p, w1p, alr1):
    grid = (N_PAD // R,)
    return pl.pallas_call(
        _tc_pre_body,
        grid=grid,
        in_specs=[
            pl.BlockSpec((R, IN_FEATS), lambda i: (i, 0)),
            pl.BlockSpec((F1E, IN_FEATS), lambda i: (0, 0)),
            pl.BlockSpec((F1E, 2), lambda i: (0, 0)),
        ],
        out_specs=[
            pl.BlockSpec((R, F1E), lambda i: (i, 0)),
            pl.BlockSpec((R, LANES), lambda i: (i, 0)),
            pl.BlockSpec((R, LANES), lambda i: (i, 0)),
            pl.BlockSpec((1, 2), lambda i: (0, 0)),
        ],
        out_shape=[
            jax.ShapeDtypeStruct((N_PAD, F1E), jnp.float32),
            jax.ShapeDtypeStruct((N_PAD, LANES), jnp.float32),
            jax.ShapeDtypeStruct((N_PAD, LANES), jnp.float32),
            jax.ShapeDtypeStruct((1, 2), jnp.float32),
        ],
    )(feat_p, w1p, alr1)


def _tc_mid_body(ua_ref, ub_ref, b1_ref, w2_ref, alr_ref, hx_ref, elr_ref,
                 err_ref, mm_ref):
    i = pl.program_id(0)
    u = ua_ref[...] + ub_ref[...]
    denom = jnp.maximum(u[:, H1:H1 + 1], 1e-9)
    rst = u[:, :H1] / denom + b1_ref[...]
    x = _elu(_elu(rst))
    h = lax.dot_general(x, w2_ref[...], (((1,), (1,)), ((), ())),
                        preferred_element_type=jnp.float32)
    lane = lax.broadcasted_iota(jnp.int32, h.shape, 1)
    h = h + jnp.where(lane == H2, 1.0, 0.0)
    hx_ref[...] = h
    eler = lax.dot_general(h, alr_ref[...], (((1,), (0,)), ((), ())),
                           preferred_element_type=jnp.float32)
    elr_ref[...] = jnp.broadcast_to(eler[:, 0:1], (h.shape[0], LANES))
    err_ref[...] = jnp.broadcast_to(eler[:, 1:2], (h.shape[0], LANES))
    mblk = jnp.max(eler, axis=0, keepdims=True)

    @pl.when(i == 0)
    def _():
        mm_ref[...] = mblk

    @pl.when(i > 0)
    def _():
        mm_ref[...] = jnp.maximum(mm_ref[...], mblk)


def _tc_mid(ua, ub, b1, w2p, alr2):
    grid = (N_PAD // R,)
    return pl.pallas_call(
        _tc_mid_body,
        grid=grid,
        in_specs=[
            pl.BlockSpec((R, F1E), lambda i: (i, 0)),
            pl.BlockSpec((R, F1E), lambda i: (i, 0)),
            pl.BlockSpec((1, H1), lambda i: (0, 0)),
            pl.BlockSpec((F2E, H1), lambda i: (0, 0)),
            pl.BlockSpec((F2E, 2), lambda i: (0, 0)),
        ],
        out_specs=[
            pl.BlockSpec((R, F2E), lambda i: (i, 0)),
            pl.BlockSpec((R, LANES), lambda i: (i, 0)),
            pl.BlockSpec((R, LANES), lambda i: (i, 0)),
            pl.BlockSpec((1, 2), lambda i: (0, 0)),
        ],
        out_shape=[
            jax.ShapeDtypeStruct((N_PAD, F2E), jnp.float32),
            jax.ShapeDtypeStruct((N_PAD, LANES), jnp.float32),
            jax.ShapeDtypeStruct((N_PAD, LANES), jnp.float32),
            jax.ShapeDtypeStruct((1, 2), jnp.float32),
        ],
    )(ua, ub, b1, w2p, alr2)


def _tc_final_body(ua_ref, ub_ref, b2_ref, wl_ref, bl_ref, us_ref, sp_ref,
                   scal_ref, out_ref):
    u = ua_ref[...] + ub_ref[...]
    denom = jnp.maximum(u[:, H2:H2 + 1], 1e-9)
    x = _elu(u[:, :H2] / denom + b2_ref[...])
    zt = lax.dot_general(wl_ref[...], x, (((1,), (1,)), ((), ())),
                         preferred_element_type=jnp.float32)
    zt = zt + bl_ref[...]
    sig = 1.0 / (1.0 + jnp.exp(-zt))
    alpha0 = scal_ref[0:1, 0:1]
    beta0 = scal_ref[0:1, 1:2]
    gamma0 = scal_ref[0:1, 2:3]
    dt = scal_ref[0:1, 3:4]
    beta = sig[0:1, :] * beta0
    gamma = sig[1:2, :] * gamma0
    alphas = sig[2:3, :] * alpha0
    us = us_ref[...]
    sp = sp_ref[...]
    up_out = us + (alphas - beta * us) * dt
    sp_out = sp + (beta * us - gamma * sp) * dt
    zero3 = jnp.zeros((3, up_out.shape[1]), jnp.float32)
    out_ref[...] = jnp.concatenate([up_out, sp_out, alphas, beta, gamma, zero3], 0)


def _tc_final(ua, ub, b2, wlp, blp, us, sp, scal):
    grid = (N_PAD // R,)
    return pl.pallas_call(
        _tc_final_body,
        grid=grid,
        in_specs=[
            pl.BlockSpec((R, F2E), lambda i: (i, 0)),
            pl.BlockSpec((R, F2E), lambda i: (i, 0)),
            pl.BlockSpec((1, H2), lambda i: (0, 0)),
            pl.BlockSpec((8, H2), lambda i: (0, 0)),
            pl.BlockSpec((8, 1), lambda i: (0, 0)),
            pl.BlockSpec((1, R), lambda i: (0, i)),
            pl.BlockSpec((1, R), lambda i: (0, i)),
            pl.BlockSpec((1, 4), lambda i: (0, 0)),
        ],
        out_specs=[pl.BlockSpec((8, R), lambda i: (0, i))],
        out_shape=[jax.ShapeDtypeStruct((8, N_PAD), jnp.float32)],
    )(ua, ub, b2, wlp, blp, us, sp, scal)


def _make_sc_edge(fext, k, cpt):
    mesh = plsc.VectorSubcoreMesh(core_axis_name="c", subcore_axis_name="s")
    rpt = N_PAD // NS

    def body(hx_hbm, elr_hbm, err_hbm, mvec_hbm, srcp_hbm, dstp_hbm, out_hbm,
             src_v, dst_v, m_v, el0_v, el1_v, er0_v, er1_v, rows0_v, rows1_v,
             zero_v, u_sh, gsem0, gsem1, ssem0, ssem1, zsem):
        cid = lax.axis_index("c")
        sid = lax.axis_index("s")
        wid = sid * NC + cid
        el_bufs = (el0_v, el1_v)
        er_bufs = (er0_v, er1_v)
        rows_bufs = (rows0_v, rows1_v)
        gsems = (gsem0, gsem1)
        ssems = (ssem0, ssem1)

        zrow = jnp.zeros((LANES,), jnp.float32)

        def zb_row(zi, _):
            def zcol(qi, _):
                zero_v[zi, pl.ds(qi * LANES, LANES)] = zrow
                return 0
            lax.fori_loop(0, fext // LANES, zcol, 0)
            return 0
        lax.fori_loop(0, ZROWS, zb_row, 0)

        nz = rpt // ZROWS
        def zdma(ji, _):
            pltpu.async_copy(zero_v, u_sh.at[pl.ds(sid * rpt + ji * ZROWS, ZROWS)],
                             zsem)
            return 0
        lax.fori_loop(0, nz, zdma, 0)

        pltpu.sync_copy(mvec_hbm, m_v)
        pltpu.sync_copy(srcp_hbm.at[wid], src_v)
        pltpu.sync_copy(dstp_hbm.at[wid], dst_v)

        def zdrain(ji, _):
            pltpu.make_async_copy(
                zero_v, u_sh.at[pl.ds(sid * rpt, ZROWS)], zsem).wait()
            return 0
        lax.fori_loop(0, nz, zdrain, 0)
        plsc.subcore_barrier()

        mvec = m_v[...]

        def gissue(ci, b):
            pltpu.async_copy(hx_hbm.at[src_v.at[ci]], rows_bufs[b], gsems[b])
            pltpu.async_copy(elr_hbm.at[src_v.at[ci]], el_bufs[b], gsems[b])
            pltpu.async_copy(err_hbm.at[dst_v.at[ci]], er_bufs[b], gsems[b])

        def gwait(b):
            pltpu.make_async_copy(hx_hbm.at[src_v.at[0]], rows_bufs[b],
                                  gsems[b]).wait()
            pltpu.make_async_copy(elr_hbm.at[src_v.at[0]], el_bufs[b],
                                  gsems[b]).wait()
            pltpu.make_async_copy(err_hbm.at[dst_v.at[0]], er_bufs[b],
                                  gsems[b]).wait()

        def sissue(ci, b):
            pltpu.async_copy(rows_bufs[b], u_sh.at[dst_v.at[ci]], ssems[b],
                             add=True)

        def swait(b):
            pltpu.make_async_copy(rows_bufs[b], u_sh.at[dst_v.at[0]],
                                  ssems[b]).wait()

        gissue(0, 0)

        def outer(g, _):
            for b in range(NBUF):
                c = g * NBUF + b
                buf = rows_bufs[b]
                elb = el_bufs[b]
                erb = er_bufs[b]

                @pl.when(c >= 1)
                def _():
                    swait(b ^ 1)

                @pl.when(c + 1 < cpt)
                def _():
                    gissue(c + 1, b ^ 1)
                gwait(b)

                def scale8(rg, _):
                    for rr in range(8):
                        ri = rg * 8 + rr
                        x = elb[ri, pl.ds(0, LANES)] + erb[ri, pl.ds(0, LANES)]
                        e = jnp.where(x >= 0, x, x * 0.2)
                        ee = jnp.exp(e - mvec)
                        for q in range(fext // LANES):
                            buf[ri, pl.ds(q * LANES, LANES)] = (
                                buf[ri, pl.ds(q * LANES, LANES)] * ee)
                    return 0
                lax.fori_loop(0, k // 8, scale8, 0)
                sissue(c, b)
            return 0
        lax.fori_loop(0, cpt // NBUF, outer, 0)
        swait((cpt - 1) % NBUF)

        plsc.subcore_barrier()
        pltpu.sync_copy(u_sh.at[pl.ds(sid * rpt, rpt)],
                        out_hbm.at[cid, pl.ds(sid * rpt, rpt)])

    return functools.partial(
        pl.kernel,
        out_type=[jax.ShapeDtypeStruct((NC, N_PAD, fext), jnp.float32)],
        mesh=mesh,
        compiler_params=pltpu.CompilerParams(
            needs_layout_passes=False, use_tc_tiling_on_sc=False),
        scratch_types=[
            pltpu.VMEM((cpt, k), jnp.int32),
            pltpu.VMEM((cpt, k), jnp.int32),
            pltpu.VMEM((LANES,), jnp.float32),
            pltpu.VMEM((k, LANES), jnp.float32),
            pltpu.VMEM((k, LANES), jnp.float32),
            pltpu.VMEM((k, LANES), jnp.float32),
            pltpu.VMEM((k, LANES), jnp.float32),
            pltpu.VMEM((k, fext), jnp.float32),
            pltpu.VMEM((k, fext), jnp.float32),
            pltpu.VMEM((ZROWS, fext), jnp.float32),
            pltpu.VMEM_SHARED((N_PAD, fext), jnp.float32),
            pltpu.SemaphoreType.DMA,
            pltpu.SemaphoreType.DMA,
            pltpu.SemaphoreType.DMA,
            pltpu.SemaphoreType.DMA,
            pltpu.SemaphoreType.DMA,
        ],
    )(body)


_sc_edge_1 = _make_sc_edge(F1E, K1, CPT1)
_sc_edge_2 = _make_sc_edge(F2E, K2, CPT2)


def kernel(edge_index, feat, unsplice, splice, alpha0, beta0, gamma0, dt,
           W1, b1, al1, ar1, W2, b2, al2, ar2, Wl, bl):
    f32 = jnp.float32
    src = edge_index[0]
    dst = edge_index[1]
    pad_e = E_PAD - E
    src_p = jnp.concatenate([src, jnp.zeros((pad_e,), jnp.int32)])
    dst_p = jnp.concatenate([dst, jnp.full((pad_e,), N, jnp.int32)])
    srcp1 = src_p.reshape(NW, CPT1, K1)
    dstp1 = dst_p.reshape(NW, CPT1, K1)
    srcp2 = src_p.reshape(NW, CPT2, K2)
    dstp2 = dst_p.reshape(NW, CPT2, K2)

    feat_p = jnp.zeros((N_PAD, IN_FEATS), f32).at[:N].set(feat)
    w1p = jnp.zeros((F1E, IN_FEATS), f32).at[:H1].set(W1)
    alr1 = jnp.zeros((F1E, 2), f32).at[:H1, 0].set(al1[0]).at[:H1, 1].set(ar1[0])
    w2p = jnp.zeros((F2E, H1), f32).at[:H2].set(W2)
    alr2 = jnp.zeros((F2E, 2), f32).at[:H2, 0].set(al2[0]).at[:H2, 1].set(ar2[0])
    wlp = jnp.zeros((8, H2), f32).at[:3].set(Wl)
    blp = jnp.zeros((8, 1), f32).at[:3, 0].set(bl)
    usp = jnp.zeros((1, N_PAD), f32).at[0, :N].set(unsplice)
    spp = jnp.zeros((1, N_PAD), f32).at[0, :N].set(splice)
    scal = jnp.stack([alpha0[0], beta0[0], gamma0[0], dt[0]]).reshape(1, 4)

    hx1, elr1, err1, mm1 = _tc_pre(feat_p, w1p, alr1)
    mvec1 = jnp.full((LANES,), mm1[0, 0] + mm1[0, 1], f32)
    (u1,) = _sc_edge_1(hx1, elr1, err1, mvec1, srcp1, dstp1)

    hx2, elr2, err2, mm2 = _tc_mid(u1[0], u1[1], b1.reshape(1, H1), w2p, alr2)
    mvec2 = jnp.full((LANES,), mm2[0, 0] + mm2[0, 1], f32)
    (u2,) = _sc_edge_2(hx2, elr2, err2, mvec2, srcp2, dstp2)

    (out8,) = _tc_final(u2[0], u2[1], b2.reshape(1, H2), wlp, blp, usp, spp, scal)

    return (out8[0, :N], out8[1, :N], out8[2, :N], out8[3, :N], out8[4, :N])


# speedup vs baseline: 1.1218x; 1.1218x over previous
import functools

import jax
import jax.numpy as jnp
from jax import lax
from jax.experimental import pallas as pl
from jax.experimental.pallas import tpu as pltpu
from jax.experimental.pallas import tpu_sc as plsc

N = 10000
E = 160000
IN_FEATS = 128
H1 = 128
H2 = 64

NC = 2
NS = 16
NW = NC * NS
LANES = 16

N_PAD = 10240
N_UPAD = 10112
R = 512
NBUF = 2
K1, CPT1 = 80, 64
K2, CPT2 = 128, 40
EPT = K1 * CPT1
E_PAD = EPT * NW
assert K2 * CPT2 == EPT

F1E = 144
F2E = 80
ZROWS = 8
RPT = N_UPAD // NS


def _elu(x):
    return jnp.where(x > 0, x, jnp.exp(jnp.minimum(x, 0.0)) - 1.0)


def _tc_pre_body(f_ref, w_ref, alr_ref, hx_ref, eler_ref, mm_ref):
    i = pl.program_id(0)
    fext = hx_ref.shape[1]
    ones_col = fext - LANES
    h = lax.dot_general(f_ref[...], w_ref[...], (((1,), (1,)), ((), ())),
                        preferred_element_type=jnp.float32)
    lane = lax.broadcasted_iota(jnp.int32, h.shape, 1)
    h = h + jnp.where(lane == ones_col, 1.0, 0.0)
    hx_ref[...] = h
    eler = lax.dot_general(alr_ref[...], h, (((1,), (1,)), ((), ())),
                           preferred_element_type=jnp.float32)
    eler_ref[...] = eler
    mblk = jnp.max(eler, axis=1, keepdims=True)

    @pl.when(i == 0)
    def _():
        mm_ref[...] = mblk

    @pl.when(i > 0)
    def _():
        mm_ref[...] = jnp.maximum(mm_ref[...], mblk)


def _tc_pre(feat_p, w1p, alr1):
    grid = (N_PAD // R,)
    return pl.pallas_call(
        _tc_pre_body,
        grid=grid,
        in_specs=[
            pl.BlockSpec((R, IN_FEATS), lambda i: (i, 0)),
            pl.BlockSpec((F1E, IN_FEATS), lambda i: (0, 0)),
            pl.BlockSpec((2, F1E), lambda i: (0, 0)),
        ],
        out_specs=[
            pl.BlockSpec((R, F1E), lambda i: (i, 0)),
            pl.BlockSpec((2, R), lambda i: (0, i)),
            pl.BlockSpec((2, 1), lambda i: (0, 0)),
        ],
        out_shape=[
            jax.ShapeDtypeStruct((N_PAD, F1E), jnp.float32),
            jax.ShapeDtypeStruct((2, N_PAD), jnp.float32),
            jax.ShapeDtypeStruct((2, 1), jnp.float32),
        ],
    )(feat_p, w1p, alr1)


def _tc_mid_body(ua_ref, ub_ref, b1_ref, w2_ref, alr_ref, hx_ref, eler_ref,
                 mm_ref):
    i = pl.program_id(0)
    u = ua_ref[...] + ub_ref[...]
    denom = jnp.maximum(u[:, H1:H1 + 1], 1e-9)
    rst = u[:, :H1] / denom + b1_ref[...]
    x = _elu(_elu(rst))
    h = lax.dot_general(x, w2_ref[...], (((1,), (1,)), ((), ())),
                        preferred_element_type=jnp.float32)
    lane = lax.broadcasted_iota(jnp.int32, h.shape, 1)
    h = h + jnp.where(lane == H2, 1.0, 0.0)
    hx_ref[...] = h
    eler = lax.dot_general(alr_ref[...], h, (((1,), (1,)), ((), ())),
                           preferred_element_type=jnp.float32)
    row = lax.broadcasted_iota(jnp.int32, eler.shape, 1) + i * R
    eler = jnp.where(row < N_UPAD, eler, -1e30)
    eler_ref[...] = eler
    mblk = jnp.max(eler, axis=1, keepdims=True)

    @pl.when(i == 0)
    def _():
        mm_ref[...] = mblk

    @pl.when(i > 0)
    def _():
        mm_ref[...] = jnp.maximum(mm_ref[...], mblk)


def _tc_mid(ua, ub, b1, w2p, alr2):
    grid = (N_PAD // R,)
    return pl.pallas_call(
        _tc_mid_body,
        grid=grid,
        in_specs=[
            pl.BlockSpec((R, F1E), lambda i: (i, 0)),
            pl.BlockSpec((R, F1E), lambda i: (i, 0)),
            pl.BlockSpec((1, H1), lambda i: (0, 0)),
            pl.BlockSpec((F2E, H1), lambda i: (0, 0)),
            pl.BlockSpec((2, F2E), lambda i: (0, 0)),
        ],
        out_specs=[
            pl.BlockSpec((R, F2E), lambda i: (i, 0)),
            pl.BlockSpec((2, R), lambda i: (0, i)),
            pl.BlockSpec((2, 1), lambda i: (0, 0)),
        ],
        out_shape=[
            jax.ShapeDtypeStruct((N_PAD, F2E), jnp.float32),
            jax.ShapeDtypeStruct((2, N_PAD), jnp.float32),
            jax.ShapeDtypeStruct((2, 1), jnp.float32),
        ],
    )(ua, ub, b1, w2p, alr2)


def _tc_final_body(ua_ref, ub_ref, b2_ref, wl_ref, bl_ref, us_ref, sp_ref,
                   scal_ref, out_ref):
    u = ua_ref[...] + ub_ref[...]
    denom = jnp.maximum(u[:, H2:H2 + 1], 1e-9)
    x = _elu(u[:, :H2] / denom + b2_ref[...])
    zt = lax.dot_general(wl_ref[...], x, (((1,), (1,)), ((), ())),
                         preferred_element_type=jnp.float32)
    zt = zt + bl_ref[...]
    sig = 1.0 / (1.0 + jnp.exp(-zt))
    alpha0 = scal_ref[0:1, 0:1]
    beta0 = scal_ref[0:1, 1:2]
    gamma0 = scal_ref[0:1, 2:3]
    dt = scal_ref[0:1, 3:4]
    beta = sig[0:1, :] * beta0
    gamma = sig[1:2, :] * gamma0
    alphas = sig[2:3, :] * alpha0
    us = us_ref[...]
    sp = sp_ref[...]
    up_out = us + (alphas - beta * us) * dt
    sp_out = sp + (beta * us - gamma * sp) * dt
    zero3 = jnp.zeros((3, up_out.shape[1]), jnp.float32)
    out_ref[...] = jnp.concatenate([up_out, sp_out, alphas, beta, gamma, zero3], 0)


def _tc_final(ua, ub, b2, wlp, blp, us, sp, scal):
    grid = (N_PAD // R,)
    return pl.pallas_call(
        _tc_final_body,
        grid=grid,
        in_specs=[
            pl.BlockSpec((R, F2E), lambda i: (i, 0)),
            pl.BlockSpec((R, F2E), lambda i: (i, 0)),
            pl.BlockSpec((1, H2), lambda i: (0, 0)),
            pl.BlockSpec((8, H2), lambda i: (0, 0)),
            pl.BlockSpec((8, 1), lambda i: (0, 0)),
            pl.BlockSpec((1, R), lambda i: (0, i)),
            pl.BlockSpec((1, R), lambda i: (0, i)),
            pl.BlockSpec((1, 4), lambda i: (0, 0)),
        ],
        out_specs=[pl.BlockSpec((8, R), lambda i: (0, i))],
        out_shape=[jax.ShapeDtypeStruct((8, N_PAD), jnp.float32)],
    )(ua, ub, b2, wlp, blp, us, sp, scal)


_SC_PARAMS = pltpu.CompilerParams(
    needs_layout_passes=False, use_tc_tiling_on_sc=False)


def _sc_attention(eler_hbm, mvec_hbm, srcp_hbm, dstp_hbm, ee_hbm,
                  el_v, er_v, src_v, dst_v, m_v, ee_v):
    cid = lax.axis_index("c")
    sid = lax.axis_index("s")
    wid = sid * NC + cid
    pltpu.sync_copy(eler_hbm.at[0], el_v)
    pltpu.sync_copy(eler_hbm.at[1], er_v)
    pltpu.sync_copy(mvec_hbm, m_v)
    pltpu.sync_copy(srcp_hbm.at[wid], src_v)
    pltpu.sync_copy(dstp_hbm.at[wid], dst_v)
    mvec = m_v[...]

    def step(t, _):
        for j in range(4):
            o = t * 64 + j * LANES
            sv = src_v[pl.ds(o, LANES)]
            dv = dst_v[pl.ds(o, LANES)]
            elg = plsc.load_gather(el_v, [sv])
            erg = plsc.load_gather(er_v, [dv])
            x = elg + erg
            e = jnp.where(x >= 0, x, x * 0.2)
            ee_v[pl.ds(o, LANES)] = jnp.exp(e - mvec)
        return 0
    lax.fori_loop(0, EPT // 64, step, 0)
    pltpu.sync_copy(ee_v, ee_hbm.at[wid])


def _make_sc_attention():
    mesh = plsc.VectorSubcoreMesh(core_axis_name="c", subcore_axis_name="s")
    return functools.partial(
        pl.kernel,
        out_type=[jax.ShapeDtypeStruct((NW, EPT), jnp.float32)],
        mesh=mesh,
        compiler_params=_SC_PARAMS,
        scratch_types=[
            pltpu.VMEM((N_PAD,), jnp.float32),
            pltpu.VMEM((N_PAD,), jnp.float32),
            pltpu.VMEM((EPT,), jnp.int32),
            pltpu.VMEM((EPT,), jnp.int32),
            pltpu.VMEM((LANES,), jnp.float32),
            pltpu.VMEM((EPT,), jnp.float32),
        ],
    )(_sc_attention)


def _zero_accumulator(u_sh, zero_v, sid, fext, zsem):
    zrow = jnp.zeros((LANES,), jnp.float32)

    def zb_row(zi, _):
        def zcol(qi, _):
            zero_v[zi, pl.ds(qi * LANES, LANES)] = zrow
            return 0
        lax.fori_loop(0, fext // LANES, zcol, 0)
        return 0
    lax.fori_loop(0, ZROWS, zb_row, 0)

    nz = RPT // ZROWS
    def zdma(ji, _):
        pltpu.async_copy(zero_v, u_sh.at[pl.ds(sid * RPT + ji * ZROWS, ZROWS)],
                         zsem)
        return 0
    lax.fori_loop(0, nz, zdma, 0)
    return nz


def _zero_drain(u_sh, zero_v, sid, nz, zsem):
    def zdrain(ji, _):
        pltpu.make_async_copy(
            zero_v, u_sh.at[pl.ds(sid * RPT, ZROWS)], zsem).wait()
        return 0
    lax.fori_loop(0, nz, zdrain, 0)


def _make_sc_scatter1():
    mesh = plsc.VectorSubcoreMesh(core_axis_name="c", subcore_axis_name="s")
    fext, k, cpt = F1E, K1, CPT1

    def body(hx_hbm, ee_hbm, srcp_hbm, dstp_hbm, out_hbm,
             src_v, dst_v, ee_v, rows0_v, rows1_v, zero_v, u_sh,
             gsem0, gsem1, ssem0, ssem1, zsem):
        cid = lax.axis_index("c")
        sid = lax.axis_index("s")
        wid = sid * NC + cid
        rows_bufs = (rows0_v, rows1_v)
        gsems = (gsem0, gsem1)
        ssems = (ssem0, ssem1)

        nz = _zero_accumulator(u_sh, zero_v, sid, fext, zsem)
        pltpu.sync_copy(srcp_hbm.at[wid], src_v)
        pltpu.sync_copy(dstp_hbm.at[wid], dst_v)
        pltpu.sync_copy(ee_hbm.at[wid], ee_v.at[pl.ds(0, EPT)])
        _zero_drain(u_sh, zero_v, sid, nz, zsem)
        plsc.subcore_barrier()

        def gissue(ci, b):
            pltpu.async_copy(hx_hbm.at[src_v.at[ci]], rows_bufs[b], gsems[b])

        def gwait(b):
            pltpu.make_async_copy(hx_hbm.at[src_v.at[0]], rows_bufs[b],
                                  gsems[b]).wait()

        def sissue(ci, b):
            pltpu.async_copy(rows_bufs[b], u_sh.at[dst_v.at[ci]], ssems[b],
                             add=True)

        def swait(b):
            pltpu.make_async_copy(rows_bufs[b], u_sh.at[dst_v.at[0]],
                                  ssems[b]).wait()

        gissue(0, 0)

        def outer(g, _):
            for b in range(NBUF):
                c = g * NBUF + b
                buf = rows_bufs[b]

                @pl.when(c >= 1)
                def _():
                    swait(b ^ 1)

                @pl.when(c + 1 < cpt)
                def _():
                    gissue(c + 1, b ^ 1)
                gwait(b)

                def scale4(rg, _):
                    base = c * k + rg * 4
                    for rr in range(4):
                        ri = rg * 4 + rr
                        ee = plsc.load_gather(
                            ee_v, [jnp.full((LANES,), base + rr, jnp.int32)])
                        vals = [buf[ri, pl.ds(q * LANES, LANES)]
                                for q in range(fext // LANES)]
                        vals = [v * ee for v in vals]
                        for q in range(fext // LANES):
                            buf[ri, pl.ds(q * LANES, LANES)] = vals[q]
                    return 0
                lax.fori_loop(0, k // 4, scale4, 0)
                sissue(c, b)
            return 0
        lax.fori_loop(0, cpt // NBUF, outer, 0)
        swait((cpt - 1) % NBUF)

        plsc.subcore_barrier()
        pltpu.sync_copy(u_sh.at[pl.ds(sid * RPT, RPT)],
                        out_hbm.at[cid, pl.ds(sid * RPT, RPT)])

    return functools.partial(
        pl.kernel,
        out_type=[jax.ShapeDtypeStruct((NC, N_PAD, fext), jnp.float32)],
        mesh=mesh,
        compiler_params=_SC_PARAMS,
        scratch_types=[
            pltpu.VMEM((cpt, k), jnp.int32),
            pltpu.VMEM((cpt, k), jnp.int32),
            pltpu.VMEM((EPT + LANES,), jnp.float32),
            pltpu.VMEM((k, fext), jnp.float32),
            pltpu.VMEM((k, fext), jnp.float32),
            pltpu.VMEM((ZROWS, fext), jnp.float32),
            pltpu.VMEM_SHARED((N_UPAD, fext), jnp.float32),
            pltpu.SemaphoreType.DMA,
            pltpu.SemaphoreType.DMA,
            pltpu.SemaphoreType.DMA,
            pltpu.SemaphoreType.DMA,
            pltpu.SemaphoreType.DMA,
        ],
    )(body)


def _make_sc_edge2():
    mesh = plsc.VectorSubcoreMesh(core_axis_name="c", subcore_axis_name="s")
    fext, k, cpt = F2E, K2, CPT2

    def body(hx_hbm, eler_hbm, mvec_hbm, srcp_hbm, dstp_hbm, out_hbm,
             el_v, er_v, src_v, dst_v, m_v, ee_v, rows0_v, rows1_v, zero_v,
             u_sh, gsem0, gsem1, ssem0, ssem1, zsem):
        cid = lax.axis_index("c")
        sid = lax.axis_index("s")
        wid = sid * NC + cid
        rows_bufs = (rows0_v, rows1_v)
        gsems = (gsem0, gsem1)
        ssems = (ssem0, ssem1)

        nz = _zero_accumulator(u_sh, zero_v, sid, fext, zsem)
        pltpu.sync_copy(eler_hbm.at[0], el_v)
        pltpu.sync_copy(eler_hbm.at[1], er_v)
        pltpu.sync_copy(mvec_hbm, m_v)
        pltpu.sync_copy(srcp_hbm.at[wid], src_v)
        pltpu.sync_copy(dstp_hbm.at[wid], dst_v)
        _zero_drain(u_sh, zero_v, sid, nz, zsem)
        plsc.subcore_barrier()
        mvec = m_v[...]

        def gissue(ci, b):
            pltpu.async_copy(hx_hbm.at[src_v.at[ci]], rows_bufs[b], gsems[b])

        def gwait(b):
            pltpu.make_async_copy(hx_hbm.at[src_v.at[0]], rows_bufs[b],
                                  gsems[b]).wait()

        def sissue(ci, b):
            pltpu.async_copy(rows_bufs[b], u_sh.at[dst_v.at[ci]], ssems[b],
                             add=True)

        def swait(b):
            pltpu.make_async_copy(rows_bufs[b], u_sh.at[dst_v.at[0]],
                                  ssems[b]).wait()

        gissue(0, 0)

        def outer(g, _):
            for b in range(NBUF):
                c = g * NBUF + b
                buf = rows_bufs[b]

                @pl.when(c >= 1)
                def _():
                    swait(b ^ 1)

                @pl.when(c + 1 < cpt)
                def _():
                    gissue(c + 1, b ^ 1)

                for j in range(k // LANES):
                    sv = src_v[c, pl.ds(j * LANES, LANES)]
                    dv = dst_v[c, pl.ds(j * LANES, LANES)]
                    elg = plsc.load_gather(el_v, [sv])
                    erg = plsc.load_gather(er_v, [dv])
                    x = elg + erg
                    e = jnp.where(x >= 0, x, x * 0.2)
                    ee_v[pl.ds(j * LANES, LANES)] = jnp.exp(e - mvec)
                gwait(b)

                def scale4(rg, _):
                    for rr in range(4):
                        ri = rg * 4 + rr
                        ee = plsc.load_gather(
                            ee_v, [jnp.full((LANES,), ri, jnp.int32)])
                        vals = [buf[ri, pl.ds(q * LANES, LANES)]
                                for q in range(fext // LANES)]
                        vals = [v * ee for v in vals]
                        for q in range(fext // LANES):
                            buf[ri, pl.ds(q * LANES, LANES)] = vals[q]
                    return 0
                lax.fori_loop(0, k // 4, scale4, 0)
                sissue(c, b)
            return 0
        lax.fori_loop(0, cpt // NBUF, outer, 0)
        swait((cpt - 1) % NBUF)

        plsc.subcore_barrier()
        pltpu.sync_copy(u_sh.at[pl.ds(sid * RPT, RPT)],
                        out_hbm.at[cid, pl.ds(sid * RPT, RPT)])

    return functools.partial(
        pl.kernel,
        out_type=[jax.ShapeDtypeStruct((NC, N_PAD, fext), jnp.float32)],
        mesh=mesh,
        compiler_params=_SC_PARAMS,
        scratch_types=[
            pltpu.VMEM((N_PAD,), jnp.float32),
            pltpu.VMEM((N_PAD,), jnp.float32),
            pltpu.VMEM((cpt, k), jnp.int32),
            pltpu.VMEM((cpt, k), jnp.int32),
            pltpu.VMEM((LANES,), jnp.float32),
            pltpu.VMEM((k + LANES,), jnp.float32),
            pltpu.VMEM((k, fext), jnp.float32),
            pltpu.VMEM((k, fext), jnp.float32),
            pltpu.VMEM((ZROWS, fext), jnp.float32),
            pltpu.VMEM_SHARED((N_UPAD, fext), jnp.float32),
            pltpu.SemaphoreType.DMA,
            pltpu.SemaphoreType.DMA,
            pltpu.SemaphoreType.DMA,
            pltpu.SemaphoreType.DMA,
            pltpu.SemaphoreType.DMA,
        ],
    )(body)


_sc_att_1 = _make_sc_attention()
_sc_scatter_1 = _make_sc_scatter1()
_sc_edge_2 = _make_sc_edge2()


def kernel(edge_index, feat, unsplice, splice, alpha0, beta0, gamma0, dt,
           W1, b1, al1, ar1, W2, b2, al2, ar2, Wl, bl):
    f32 = jnp.float32
    src = edge_index[0]
    dst = edge_index[1]
    pad_e = E_PAD - E
    src_p = jnp.concatenate([src, jnp.zeros((pad_e,), jnp.int32)])
    dst_p = jnp.concatenate([dst, jnp.full((pad_e,), N, jnp.int32)])
    srcp1 = src_p.reshape(NW, CPT1, K1)
    dstp1 = dst_p.reshape(NW, CPT1, K1)
    srcp1f = src_p.reshape(NW, EPT)
    dstp1f = dst_p.reshape(NW, EPT)
    srcp2 = src_p.reshape(NW, CPT2, K2)
    dstp2 = dst_p.reshape(NW, CPT2, K2)

    feat_p = jnp.zeros((N_PAD, IN_FEATS), f32).at[:N].set(feat)
    w1p = jnp.zeros((F1E, IN_FEATS), f32).at[:H1].set(W1)
    alr1 = jnp.zeros((2, F1E), f32).at[0, :H1].set(al1[0]).at[1, :H1].set(ar1[0])
    w2p = jnp.zeros((F2E, H1), f32).at[:H2].set(W2)
    alr2 = jnp.zeros((2, F2E), f32).at[0, :H2].set(al2[0]).at[1, :H2].set(ar2[0])
    wlp = jnp.zeros((8, H2), f32).at[:3].set(Wl)
    blp = jnp.zeros((8, 1), f32).at[:3, 0].set(bl)
    usp = jnp.zeros((1, N_PAD), f32).at[0, :N].set(unsplice)
    spp = jnp.zeros((1, N_PAD), f32).at[0, :N].set(splice)
    scal = jnp.stack([alpha0[0], beta0[0], gamma0[0], dt[0]]).reshape(1, 4)

    hx1, eler1, mm1 = _tc_pre(feat_p, w1p, alr1)
    mvec1 = jnp.full((LANES,), mm1[0, 0] + mm1[1, 0], f32)
    (ee1,) = _sc_att_1(eler1, mvec1, srcp1f, dstp1f)
    (u1,) = _sc_scatter_1(hx1, ee1, srcp1, dstp1)

    hx2, eler2, mm2 = _tc_mid(u1[0], u1[1], b1.reshape(1, H1), w2p, alr2)
    mvec2 = jnp.full((LANES,), mm2[0, 0] + mm2[1, 0], f32)
    (u2,) = _sc_edge_2(hx2, eler2, mvec2, srcp2, dstp2)

    (out8,) = _tc_final(u2[0], u2[1], b2.reshape(1, H2), wlp, blp, usp, spp, scal)

    return (out8[0, :N], out8[1, :N], out8[2, :N], out8[3, :N], out8[4, :N])
